# Initial kernel scaffold; baseline (speedup 1.0000x reference)
import jax
import jax.numpy as jnp
from jax.experimental import pallas as pl


def kernel(hidden_states, weight, expert_biases):
    raise NotImplementedError("write your pallas kernel here")


# speedup vs baseline: 1.4949x; 1.4949x over previous
import functools

import jax
import jax.numpy as jnp
from jax import lax
from jax.experimental import pallas as pl

TOPK = 8
NUM_EXPERTS = 64


def _gate_block(x_ref, w_ref, b_ref, idx_ref, wgt_ref):
    x = x_ref[...]
    w = w_ref[...]
    logits = lax.dot_general(x, w, (((1,), (1,)), ((), ())),
                             preferred_element_type=jnp.float32)
    biased = logits + b_ref[...]
    rows = biased.shape[0]
    iota = lax.broadcasted_iota(jnp.int32, (rows, NUM_EXPERTS), 1)
    remaining = biased
    idx_cols = []
    val_cols = []
    for _ in range(TOPK):
        m = jnp.max(remaining, axis=1, keepdims=True)
        eq = remaining == m
        idx = jnp.min(jnp.where(eq, iota, NUM_EXPERTS), axis=1, keepdims=True)
        chosen = iota == idx
        val = jnp.max(jnp.where(chosen, logits, -jnp.inf), axis=1,
                      keepdims=True)
        idx_cols.append(idx)
        val_cols.append(val)
        remaining = jnp.where(chosen, -jnp.inf, remaining)
    top_idx = jnp.concatenate(idx_cols, axis=1)
    top_val = jnp.concatenate(val_cols, axis=1)
    probs = jax.nn.sigmoid(top_val)
    denom = jnp.maximum(jnp.sum(jnp.abs(probs), axis=1, keepdims=True), 1e-12)
    idx_ref[...] = top_idx
    wgt_ref[...] = probs / denom


@functools.partial(jax.jit, static_argnames=("block_rows",))
def _gate(x, weight, bias2d, block_rows=512):
    n = x.shape[0]
    grid = (n // block_rows,)
    out_shape = (
        jax.ShapeDtypeStruct((n, TOPK), jnp.int32),
        jax.ShapeDtypeStruct((n, TOPK), jnp.float32),
    )
    return pl.pallas_call(
        _gate_block,
        grid=grid,
        in_specs=[
            pl.BlockSpec((block_rows, x.shape[1]), lambda i: (i, 0)),
            pl.BlockSpec(weight.shape, lambda i: (0, 0)),
            pl.BlockSpec(bias2d.shape, lambda i: (0, 0)),
        ],
        out_specs=(
            pl.BlockSpec((block_rows, TOPK), lambda i: (i, 0)),
            pl.BlockSpec((block_rows, TOPK), lambda i: (i, 0)),
        ),
        out_shape=out_shape,
    )(x, weight, bias2d)


def kernel(hidden_states, weight, expert_biases):
    bsz, seq_len, h = hidden_states.shape
    x = hidden_states.reshape(-1, h)
    idx, wgt = _gate(x, weight, expert_biases.reshape(1, NUM_EXPERTS))
    return idx.reshape(bsz, seq_len, TOPK), wgt.reshape(bsz, seq_len, TOPK)
